# baseline (device time: 156302 ns/iter reference)
import jax
import jax.numpy as jnp
from jax import lax
from jax.experimental import pallas as pl
from jax.experimental.pallas import tpu as pltpu

N_DEV = 4
C = 8


def kernel(x, w_mat):
    m_per, k = x.shape
    _, n_per = w_mat.shape
    half = m_per // 2
    sub = half // C
    n_hops = N_DEV - 1
    n_slots = n_hops * C

    x = x.astype(jnp.bfloat16)

    def body(x_ref, w_hbm, out_ref, w_ref,
             cw_ref, ccw_ref, amax_ref, w_sem,
             cw_send, cw_recv, ccw_send, ccw_recv, a_send, a_recv):
        my = lax.axis_index("i")
        left = (my + N_DEV - 1) % N_DEV
        right = (my + 1) % N_DEV

        barrier_sem = pltpu.get_barrier_semaphore()
        for nbr in (left, right):
            pl.semaphore_signal(
                barrier_sem, inc=1,
                device_id=(nbr,), device_id_type=pl.DeviceIdType.MESH,
            )
        pl.semaphore_wait(barrier_sem, 2)

        w_copy = pltpu.make_async_copy(w_hbm, out_ref, w_sem)
        w_copy.start()

        def gemm_into(chunk, row0, rows):
            y = jnp.dot(chunk, w_ref[...],
                        preferred_element_type=jnp.float32)
            y = jnp.maximum(y, 0.0)
            out_ref[pl.ds(row0, rows), :] = y
            return jnp.max(y)

        def slot(h, c):
            return h * C + c

        def mk(src, dst, send_sems, recv_sems, h, c, dev):
            return pltpu.make_async_remote_copy(
                src_ref=src,
                dst_ref=dst.at[slot(h, c)],
                send_sem=send_sems.at[slot(h, c)],
                recv_sem=recv_sems.at[slot(h, c)],
                device_id=(dev,),
                device_id_type=pl.DeviceIdType.MESH,
            )

        cw_rdmas = {}
        ccw_rdmas = {}
        for c in range(C):
            r = mk(x_ref.at[pl.ds(c * sub, sub)], cw_ref,
                   cw_send, cw_recv, 0, c, right)
            r.start()
            cw_rdmas[(0, c)] = r
            r = mk(x_ref.at[pl.ds(half + c * sub, sub)], ccw_ref,
                   ccw_send, ccw_recv, 0, c, left)
            r.start()
            ccw_rdmas[(0, c)] = r

        w_copy.wait()
        w_ref[...] = out_ref[...].astype(jnp.bfloat16)

        amax = gemm_into(x_ref[...], my * m_per, m_per)

        for h in range(1, n_hops):
            for c in range(C):
                cw_rdmas[(h - 1, c)].wait_recv()
                r = mk(cw_ref.at[slot(h - 1, c)], cw_ref,
                       cw_send, cw_recv, h, c, right)
                r.start()
                cw_rdmas[(h, c)] = r
                ccw_rdmas[(h - 1, c)].wait_recv()
                r = mk(ccw_ref.at[slot(h - 1, c)], ccw_ref,
                       ccw_send, ccw_recv, h, c, left)
                r.start()
                ccw_rdmas[(h, c)] = r
            cw_org = (my + N_DEV - h) % N_DEV
            ccw_org = (my + h) % N_DEV
            for c in range(C):
                amax = jnp.maximum(amax, gemm_into(
                    cw_ref[slot(h - 1, c)], cw_org * m_per + c * sub, sub))
                amax = jnp.maximum(amax, gemm_into(
                    ccw_ref[slot(h - 1, c)],
                    ccw_org * m_per + half + c * sub, sub))

        cw_org = (my + 1) % N_DEV
        ccw_org = left
        for c in range(C):
            cw_rdmas[(n_hops - 1, c)].wait_recv()
            amax = jnp.maximum(amax, gemm_into(
                cw_ref[slot(n_hops - 1, c)], cw_org * m_per + c * sub, sub))
            ccw_rdmas[(n_hops - 1, c)].wait_recv()
            amax = jnp.maximum(amax, gemm_into(
                ccw_ref[slot(n_hops - 1, c)],
                ccw_org * m_per + half + c * sub, sub))

        amax_ref[N_DEV - 1] = jnp.full((8, 128), amax, jnp.float32)
        a_rdmas = []
        for kk in range(1, N_DEV):
            r = pltpu.make_async_remote_copy(
                src_ref=amax_ref.at[N_DEV - 1],
                dst_ref=amax_ref.at[kk - 1],
                send_sem=a_send.at[kk - 1],
                recv_sem=a_recv.at[kk - 1],
                device_id=((my + kk) % N_DEV,),
                device_id_type=pl.DeviceIdType.MESH,
            )
            r.start()
            a_rdmas.append(r)

        for r in cw_rdmas.values():
            r.wait_send()
        for r in ccw_rdmas.values():
            r.wait_send()

        for r in a_rdmas:
            r.wait()
        gmax = amax
        for s in range(N_DEV - 1):
            gmax = jnp.maximum(gmax, amax_ref[s, 0, 0])

        scale = gmax / 448.0
        inv = 448.0 / gmax
        q = jnp.minimum(out_ref[...] * inv, 448.0).astype(jnp.float8_e4m3fn)
        out_ref[...] = q.astype(jnp.float32) * scale

    return pl.pallas_call(
        body,
        out_shape=jax.ShapeDtypeStruct((N_DEV * m_per, n_per), jnp.float32),
        in_specs=[
            pl.BlockSpec(memory_space=pltpu.VMEM),
            pl.BlockSpec(memory_space=pltpu.MemorySpace.HBM),
        ],
        out_specs=pl.BlockSpec(memory_space=pltpu.VMEM),
        scratch_shapes=[
            pltpu.VMEM((k, n_per), jnp.bfloat16),
            pltpu.VMEM((n_slots, sub, k), jnp.bfloat16),
            pltpu.VMEM((n_slots, sub, k), jnp.bfloat16),
            pltpu.VMEM((N_DEV, 8, 128), jnp.float32),
            pltpu.SemaphoreType.DMA,
            pltpu.SemaphoreType.DMA((n_slots,)),
            pltpu.SemaphoreType.DMA((n_slots,)),
            pltpu.SemaphoreType.DMA((n_slots,)),
            pltpu.SemaphoreType.DMA((n_slots,)),
            pltpu.SemaphoreType.DMA((N_DEV - 1,)),
            pltpu.SemaphoreType.DMA((N_DEV - 1,)),
        ],
        compiler_params=pltpu.CompilerParams(collective_id=0),
    )(x, w_mat)


# device time: 155840 ns/iter; 1.0030x vs baseline; 1.0030x over previous
import jax
import jax.numpy as jnp
from jax import lax
from jax.experimental import pallas as pl
from jax.experimental.pallas import tpu as pltpu

N_DEV = 4
C = 4


def kernel(x, w_mat):
    m_per, k = x.shape
    _, n_per = w_mat.shape
    half = m_per // 2
    sub = half // C
    n_hops = N_DEV - 1
    n_slots = n_hops * C

    x = x.astype(jnp.bfloat16)

    def body(x_ref, w_hbm, out_ref, w_ref,
             cw_ref, ccw_ref, amax_ref, w_sem,
             cw_send, cw_recv, ccw_send, ccw_recv, a_send, a_recv):
        my = lax.axis_index("i")
        left = (my + N_DEV - 1) % N_DEV
        right = (my + 1) % N_DEV

        barrier_sem = pltpu.get_barrier_semaphore()
        for nbr in (left, right):
            pl.semaphore_signal(
                barrier_sem, inc=1,
                device_id=(nbr,), device_id_type=pl.DeviceIdType.MESH,
            )
        pl.semaphore_wait(barrier_sem, 2)

        w_copy = pltpu.make_async_copy(w_hbm, out_ref, w_sem)
        w_copy.start()

        def gemm_into(chunk, row0, rows):
            y = jnp.dot(chunk, w_ref[...],
                        preferred_element_type=jnp.float32)
            y = jnp.maximum(y, 0.0)
            out_ref[pl.ds(row0, rows), :] = y
            return jnp.max(y)

        def slot(h, c):
            return h * C + c

        def mk(src, dst, send_sems, recv_sems, h, c, dev):
            return pltpu.make_async_remote_copy(
                src_ref=src,
                dst_ref=dst.at[slot(h, c)],
                send_sem=send_sems.at[slot(h, c)],
                recv_sem=recv_sems.at[slot(h, c)],
                device_id=(dev,),
                device_id_type=pl.DeviceIdType.MESH,
            )

        cw_rdmas = {}
        ccw_rdmas = {}
        for c in range(C):
            r = mk(x_ref.at[pl.ds(c * sub, sub)], cw_ref,
                   cw_send, cw_recv, 0, c, right)
            r.start()
            cw_rdmas[(0, c)] = r
            r = mk(x_ref.at[pl.ds(half + c * sub, sub)], ccw_ref,
                   ccw_send, ccw_recv, 0, c, left)
            r.start()
            ccw_rdmas[(0, c)] = r

        w_copy.wait()
        w_ref[...] = out_ref[...].astype(jnp.bfloat16)

        amax = gemm_into(x_ref[...], my * m_per, m_per)

        for h in range(1, n_hops):
            for c in range(C):
                cw_rdmas[(h - 1, c)].wait_recv()
                r = mk(cw_ref.at[slot(h - 1, c)], cw_ref,
                       cw_send, cw_recv, h, c, right)
                r.start()
                cw_rdmas[(h, c)] = r
                ccw_rdmas[(h - 1, c)].wait_recv()
                r = mk(ccw_ref.at[slot(h - 1, c)], ccw_ref,
                       ccw_send, ccw_recv, h, c, left)
                r.start()
                ccw_rdmas[(h, c)] = r
            cw_org = (my + N_DEV - h) % N_DEV
            ccw_org = (my + h) % N_DEV
            for c in range(C):
                amax = jnp.maximum(amax, gemm_into(
                    cw_ref[slot(h - 1, c)], cw_org * m_per + c * sub, sub))
                amax = jnp.maximum(amax, gemm_into(
                    ccw_ref[slot(h - 1, c)],
                    ccw_org * m_per + half + c * sub, sub))

        cw_org = (my + 1) % N_DEV
        ccw_org = left
        for c in range(C):
            cw_rdmas[(n_hops - 1, c)].wait_recv()
            amax = jnp.maximum(amax, gemm_into(
                cw_ref[slot(n_hops - 1, c)], cw_org * m_per + c * sub, sub))
            ccw_rdmas[(n_hops - 1, c)].wait_recv()
            amax = jnp.maximum(amax, gemm_into(
                ccw_ref[slot(n_hops - 1, c)],
                ccw_org * m_per + half + c * sub, sub))

        amax_ref[N_DEV - 1] = jnp.full((8, 128), amax, jnp.float32)
        a_rdmas = []
        for kk in range(1, N_DEV):
            r = pltpu.make_async_remote_copy(
                src_ref=amax_ref.at[N_DEV - 1],
                dst_ref=amax_ref.at[kk - 1],
                send_sem=a_send.at[kk - 1],
                recv_sem=a_recv.at[kk - 1],
                device_id=((my + kk) % N_DEV,),
                device_id_type=pl.DeviceIdType.MESH,
            )
            r.start()
            a_rdmas.append(r)

        for r in cw_rdmas.values():
            r.wait_send()
        for r in ccw_rdmas.values():
            r.wait_send()

        for r in a_rdmas:
            r.wait()
        gmax = amax
        for s in range(N_DEV - 1):
            gmax = jnp.maximum(gmax, amax_ref[s, 0, 0])

        scale = gmax / 448.0
        inv = 448.0 / gmax
        q = jnp.minimum(out_ref[...] * inv, 448.0).astype(jnp.float8_e4m3fn)
        out_ref[...] = q.astype(jnp.float32) * scale

    return pl.pallas_call(
        body,
        out_shape=jax.ShapeDtypeStruct((N_DEV * m_per, n_per), jnp.float32),
        in_specs=[
            pl.BlockSpec(memory_space=pltpu.VMEM),
            pl.BlockSpec(memory_space=pltpu.MemorySpace.HBM),
        ],
        out_specs=pl.BlockSpec(memory_space=pltpu.VMEM),
        scratch_shapes=[
            pltpu.VMEM((k, n_per), jnp.bfloat16),
            pltpu.VMEM((n_slots, sub, k), jnp.bfloat16),
            pltpu.VMEM((n_slots, sub, k), jnp.bfloat16),
            pltpu.VMEM((N_DEV, 8, 128), jnp.float32),
            pltpu.SemaphoreType.DMA,
            pltpu.SemaphoreType.DMA((n_slots,)),
            pltpu.SemaphoreType.DMA((n_slots,)),
            pltpu.SemaphoreType.DMA((n_slots,)),
            pltpu.SemaphoreType.DMA((n_slots,)),
            pltpu.SemaphoreType.DMA((N_DEV - 1,)),
            pltpu.SemaphoreType.DMA((N_DEV - 1,)),
        ],
        compiler_params=pltpu.CompilerParams(collective_id=0),
    )(x, w_mat)
